# baseline (speedup 1.0000x reference)
import functools

import jax
import jax.numpy as jnp
from jax import lax
from jax.experimental import pallas as pl
from jax.experimental.pallas import tpu as pltpu
from jax.experimental.pallas import tpu_sc as plsc

NC = 2
NS = 16
NW = NC * NS
LANES = 16
Q = 64


def _sc_triplane(table, qt, *, R, C, H, W, N):
    nchunk = R // Q
    iters = (nchunk + NW - 1) // NW
    B = R // N
    CW = C // 2
    mesh = plsc.VectorSubcoreMesh(
        core_axis_name="c", subcore_axis_name="s",
        num_cores=NC, num_subcores=NS)

    picks = ((0, 1), (1, 2), (2, 0))

    @functools.partial(
        pl.kernel,
        out_type=jax.ShapeDtypeStruct((R, 3 * C), jnp.float32),
        mesh=mesh,
        scratch_types=dict(
            q_v=pltpu.VMEM((2, 3, Q), jnp.float32),
            idx_v=pltpu.VMEM((2, 3, 2, Q), jnp.int32),
            w_v=pltpu.VMEM((2, 3, 4, Q), jnp.float32),
            rows=pltpu.VMEM((2, 2, Q, C), jnp.int32),
            out_v=pltpu.VMEM((2, Q, 3 * C), jnp.float32),
            sems=pltpu.SemaphoreType.DMA((2,)),
            osem=pltpu.SemaphoreType.DMA((2,)),
        ),
    )
    def k(table_h, qt_h, out_h, q_v, idx_v, w_v, rows, out_v,
          sems, osem):
        wid = lax.axis_index("s") * NC + lax.axis_index("c")

        def q_index_stage(ip, it):
            ci = wid + it * NW

            @pl.when(ci < nchunk)
            def _():
                base = ci * Q
                for c3 in range(3):
                    pltpu.sync_copy(qt_h.at[c3, pl.ds(base, Q)],
                                    q_v.at[ip, c3])
                for p in range(3):
                    px, py = picks[p]
                    for i in range(Q // LANES):
                        sl = pl.ds(i * LANES, LANES)
                        xq = q_v[ip, px, sl]
                        yq = q_v[ip, py, sl]
                        ix = jnp.minimum(jnp.maximum(
                            (xq + 1.0) * (0.5 * (W - 1)), 0.0), float(W - 1))
                        iy = jnp.minimum(jnp.maximum(
                            (yq + 1.0) * (0.5 * (H - 1)), 0.0), float(H - 1))
                        x0 = jnp.minimum(ix.astype(jnp.int32), W - 2)
                        y0 = jnp.minimum(iy.astype(jnp.int32), H - 2)
                        wx = ix - x0.astype(jnp.float32)
                        wy = iy - y0.astype(jnp.float32)
                        ux = 1.0 - wx
                        uy = 1.0 - wy
                        w_v[ip, p, 0, sl] = ux * uy
                        w_v[ip, p, 1, sl] = wx * uy
                        w_v[ip, p, 2, sl] = ux * wy
                        w_v[ip, p, 3, sl] = wx * wy
                        rowid = jnp.arange(LANES, dtype=jnp.int32) + (
                            base + i * LANES)
                        b = jnp.zeros((LANES,), jnp.int32)
                        one = jnp.ones((LANES,), jnp.int32)
                        for bb_ in range(1, B):
                            b = b + jnp.where(rowid >= bb_ * N, one, 0)
                        bi = ((b * 3 + p) * H + y0) * W + x0
                        idx_v[ip, p, 0, sl] = bi
                        idx_v[ip, p, 1, sl] = bi + W

        def fire(ip, p, rbuf):
            for y_ in range(2):
                pltpu.async_copy(table_h.at[idx_v.at[ip, p, y_]],
                                 rows.at[rbuf, y_], sems.at[rbuf])

        def wait_rows(rbuf):
            for y_ in range(2):
                pltpu.make_async_copy(table_h.at[idx_v.at[0, 0, y_]],
                                      rows.at[rbuf, y_],
                                      sems.at[rbuf]).wait()

        def blend_plane(ip, p, rbuf, ob):
            @plsc.parallel_loop(0, Q, step=1, unroll=8)
            def _blend(j):
                grp = j & -LANES
                lidx = jnp.full((LANES,), j & (LANES - 1), dtype=jnp.int32)
                himask = jnp.full((LANES,), -65536, dtype=jnp.int32)
                cw = []
                for t in range(4):
                    wg = w_v[ip, p, t, pl.ds(grp, LANES)]
                    cw.append(wg.at[lidx].get(mode="promise_in_bounds"))
                for g in range(CW // LANES):
                    f_lo = []
                    f_hi = []
                    for y_, off in ((0, 0), (0, CW), (1, 0), (1, CW)):
                        wv = rows[rbuf, y_, j, pl.ds(off + g * LANES, LANES)]
                        f_lo.append(lax.bitcast_convert_type(
                            lax.shift_left(wv, 16), jnp.float32))
                        f_hi.append(lax.bitcast_convert_type(
                            lax.bitwise_and(wv, himask), jnp.float32))
                    for half, f in ((0, f_lo), (1, f_hi)):
                        acc = (f[0] * cw[0] + f[1] * cw[1]
                               + f[2] * cw[2] + f[3] * cw[3])
                        out_v[ob, j,
                              pl.ds(p * C + (g * 2 + half) * LANES, LANES)
                              ] = acc

        def one_chunk(it, ob, bufmap):
            ci = wid + it * NW
            ipc = ob
            ipn = 1 - ob

            @pl.when(ci < nchunk)
            def _():
                base = ci * Q

                @pl.when(it >= 2)
                def _():
                    pltpu.make_async_copy(
                        out_v.at[ob], out_h.at[pl.ds(0, Q)],
                        osem.at[ob]).wait()

                wait_rows(bufmap[0])
                blend_plane(ipc, 0, bufmap[0], ob)
                fire(ipc, 2, bufmap[0])

                q_index_stage(ipn, it + 1)

                wait_rows(bufmap[1])
                blend_plane(ipc, 1, bufmap[1], ob)

                @pl.when(ci + NW < nchunk)
                def _():
                    fire(ipn, 0, bufmap[1])

                wait_rows(bufmap[2])
                blend_plane(ipc, 2, bufmap[2], ob)

                @pl.when(ci + NW < nchunk)
                def _():
                    fire(ipn, 1, bufmap[2])

                pltpu.async_copy(out_v.at[ob], out_h.at[pl.ds(base, Q)],
                                 osem.at[ob])

        def pair_body(it2, _):
            one_chunk(it2 * 2, 0, (0, 1, 0))
            one_chunk(it2 * 2 + 1, 1, (1, 0, 1))
            return 0

        q_index_stage(0, 0)

        @pl.when(wid < nchunk)
        def _():
            fire(0, 0, 0)
            fire(0, 1, 1)

        lax.fori_loop(0, (iters + 1) // 2, pair_body, 0)

        nch = lax.shift_right_logical(nchunk - wid + (NW - 1), 5)

        @pl.when(nch >= 1)
        def _():
            pltpu.make_async_copy(
                out_v.at[0], out_h.at[pl.ds(0, Q)], osem.at[0]).wait()

        @pl.when(nch >= 2)
        def _():
            pltpu.make_async_copy(
                out_v.at[1], out_h.at[pl.ds(0, Q)], osem.at[1]).wait()

    return k(table, qt)


def kernel(rolled_out_feature, query):
    B, C, H, W3 = rolled_out_feature.shape
    W = W3 // 3
    _, N, _ = query.shape
    R = B * N
    V = B * 3 * H * W
    perm = (jnp.arange(C, dtype=jnp.int32).reshape(C // 32, 2, 16)
            .transpose(0, 2, 1).reshape(C))
    table = rolled_out_feature.reshape(B, C, H, 3, W).transpose(0, 3, 2, 4, 1)
    table = table.reshape(V, C)[:, perm].astype(jnp.bfloat16)
    t32 = lax.bitcast_convert_type(
        table.reshape(V, C // 2, 2), jnp.int32)
    tpad = jnp.concatenate([t32, t32[-1:]], axis=0)
    table_ov = jnp.concatenate([tpad[:-1], tpad[1:]], axis=1)
    qt = query.reshape(R, 3).T
    out = _sc_triplane(table_ov, qt, R=R, C=C, H=H, W=W, N=N)
    return out.reshape(B, N, 3 * C)


# speedup vs baseline: 1.1839x; 1.1839x over previous
import functools

import jax
import jax.numpy as jnp
from jax import lax
from jax.experimental import pallas as pl
from jax.experimental.pallas import tpu as pltpu
from jax.experimental.pallas import tpu_sc as plsc

NC = 2
NS = 16
NW = NC * NS
LANES = 16
Q = 64


def _sc_triplane(table, q0, q1, q2, *, R, C, H, W, N):
    nchunk = R // Q
    iters = (nchunk + NW - 1) // NW
    B = R // N
    CW = C // 2
    mesh = plsc.VectorSubcoreMesh(
        core_axis_name="c", subcore_axis_name="s",
        num_cores=NC, num_subcores=NS)

    picks = ((0, 1), (1, 2), (2, 0))

    @functools.partial(
        pl.kernel,
        out_type=jax.ShapeDtypeStruct((R, 3 * C), jnp.float32),
        mesh=mesh,
        scratch_types=dict(
            q_v=pltpu.VMEM((2, 3, Q), jnp.float32),
            idx_v=pltpu.VMEM((2, 3, 2, Q), jnp.int32),
            w_v=pltpu.VMEM((2, 3, 4, Q), jnp.float32),
            rows=pltpu.VMEM((2, 2, Q, C), jnp.int32),
            out_v=pltpu.VMEM((2, Q, 3 * C), jnp.float32),
            sems=pltpu.SemaphoreType.DMA((2,)),
            osem=pltpu.SemaphoreType.DMA((2,)),
            qsem=pltpu.SemaphoreType.DMA,
        ),
    )
    def k(table_h, q0_h, q1_h, q2_h, out_h, q_v, idx_v, w_v, rows, out_v,
          sems, osem, qsem):
        wid = lax.axis_index("s") * NC + lax.axis_index("c")

        def q_index_stage(ip, it):
            ci = wid + it * NW

            @pl.when(ci < nchunk)
            def _():
                base = ci * Q
                qcps = [pltpu.async_copy(qh.at[pl.ds(base, Q)],
                                         q_v.at[ip, c3], qsem)
                        for c3, qh in enumerate((q0_h, q1_h, q2_h))]
                for qcp in qcps:
                    qcp.wait()
                for p in range(3):
                    px, py = picks[p]
                    for i in range(Q // LANES):
                        sl = pl.ds(i * LANES, LANES)
                        xq = q_v[ip, px, sl]
                        yq = q_v[ip, py, sl]
                        ix = jnp.minimum(jnp.maximum(
                            (xq + 1.0) * (0.5 * (W - 1)), 0.0), float(W - 1))
                        iy = jnp.minimum(jnp.maximum(
                            (yq + 1.0) * (0.5 * (H - 1)), 0.0), float(H - 1))
                        x0 = jnp.minimum(ix.astype(jnp.int32), W - 2)
                        y0 = jnp.minimum(iy.astype(jnp.int32), H - 2)
                        wx = ix - x0.astype(jnp.float32)
                        wy = iy - y0.astype(jnp.float32)
                        ux = 1.0 - wx
                        uy = 1.0 - wy
                        w_v[ip, p, 0, sl] = ux * uy
                        w_v[ip, p, 1, sl] = wx * uy
                        w_v[ip, p, 2, sl] = ux * wy
                        w_v[ip, p, 3, sl] = wx * wy
                        rowid = jnp.arange(LANES, dtype=jnp.int32) + (
                            base + i * LANES)
                        b = jnp.zeros((LANES,), jnp.int32)
                        one = jnp.ones((LANES,), jnp.int32)
                        for bb_ in range(1, B):
                            b = b + jnp.where(rowid >= bb_ * N, one, 0)
                        bi = ((b * 3 + p) * H + y0) * W + x0
                        idx_v[ip, p, 0, sl] = bi
                        idx_v[ip, p, 1, sl] = bi + W

        def fire(ip, p, rbuf):
            for y_ in range(2):
                pltpu.async_copy(table_h.at[idx_v.at[ip, p, y_]],
                                 rows.at[rbuf, y_], sems.at[rbuf])

        def wait_rows(rbuf):
            for y_ in range(2):
                pltpu.make_async_copy(table_h.at[idx_v.at[0, 0, y_]],
                                      rows.at[rbuf, y_],
                                      sems.at[rbuf]).wait()

        def blend_plane(ip, p, rbuf, ob):
            @plsc.parallel_loop(0, Q, step=1, unroll=8)
            def _blend(j):
                grp = j & -LANES
                lidx = jnp.full((LANES,), j & (LANES - 1), dtype=jnp.int32)
                himask = jnp.full((LANES,), -65536, dtype=jnp.int32)
                cw = []
                for t in range(4):
                    wg = w_v[ip, p, t, pl.ds(grp, LANES)]
                    cw.append(wg.at[lidx].get(mode="promise_in_bounds"))
                for g in range(CW // LANES):
                    f_lo = []
                    f_hi = []
                    for y_, off in ((0, 0), (0, CW), (1, 0), (1, CW)):
                        wv = rows[rbuf, y_, j, pl.ds(off + g * LANES, LANES)]
                        f_lo.append(lax.bitcast_convert_type(
                            lax.shift_left(wv, 16), jnp.float32))
                        f_hi.append(lax.bitcast_convert_type(
                            lax.bitwise_and(wv, himask), jnp.float32))
                    for half, f in ((0, f_lo), (1, f_hi)):
                        acc = (f[0] * cw[0] + f[1] * cw[1]
                               + f[2] * cw[2] + f[3] * cw[3])
                        out_v[ob, j,
                              pl.ds(p * C + (g * 2 + half) * LANES, LANES)
                              ] = acc

        def one_chunk(it, ob, bufmap):
            ci = wid + it * NW
            ipc = ob
            ipn = 1 - ob

            @pl.when(ci < nchunk)
            def _():
                base = ci * Q

                @pl.when(it >= 2)
                def _():
                    pltpu.make_async_copy(
                        out_v.at[ob], out_h.at[pl.ds(0, Q)],
                        osem.at[ob]).wait()

                wait_rows(bufmap[0])
                blend_plane(ipc, 0, bufmap[0], ob)
                fire(ipc, 2, bufmap[0])

                q_index_stage(ipn, it + 1)

                wait_rows(bufmap[1])
                blend_plane(ipc, 1, bufmap[1], ob)

                @pl.when(ci + NW < nchunk)
                def _():
                    fire(ipn, 0, bufmap[1])

                wait_rows(bufmap[2])
                blend_plane(ipc, 2, bufmap[2], ob)

                @pl.when(ci + NW < nchunk)
                def _():
                    fire(ipn, 1, bufmap[2])

                pltpu.async_copy(out_v.at[ob], out_h.at[pl.ds(base, Q)],
                                 osem.at[ob])

        def pair_body(it2, _):
            one_chunk(it2 * 2, 0, (0, 1, 0))
            one_chunk(it2 * 2 + 1, 1, (1, 0, 1))
            return 0

        q_index_stage(0, 0)

        @pl.when(wid < nchunk)
        def _():
            fire(0, 0, 0)
            fire(0, 1, 1)

        lax.fori_loop(0, (iters + 1) // 2, pair_body, 0)

        nch = lax.shift_right_logical(nchunk - wid + (NW - 1), 5)

        @pl.when(nch >= 1)
        def _():
            pltpu.make_async_copy(
                out_v.at[0], out_h.at[pl.ds(0, Q)], osem.at[0]).wait()

        @pl.when(nch >= 2)
        def _():
            pltpu.make_async_copy(
                out_v.at[1], out_h.at[pl.ds(0, Q)], osem.at[1]).wait()

    return k(table, q0, q1, q2)


def kernel(rolled_out_feature, query):
    B, C, H, W3 = rolled_out_feature.shape
    W = W3 // 3
    _, N, _ = query.shape
    R = B * N
    table = rolled_out_feature.reshape(B, C, H, 3, W).transpose(0, 3, 2, 4, 1)
    table = table.reshape(B * 3 * H * W, C).astype(jnp.bfloat16)
    table = table.reshape(-1, C // 32, 2, 16).transpose(0, 1, 3, 2)
    t32 = lax.bitcast_convert_type(
        table.reshape(-1, C // 2, 2), jnp.int32)
    t32next = jnp.concatenate([t32[1:], t32[:1]], axis=0)
    table_ov = jnp.concatenate([t32, t32next], axis=1)
    qf = query.reshape(R, 3)
    out = _sc_triplane(table_ov, qf[:, 0], qf[:, 1], qf[:, 2],
                       R=R, C=C, H=H, W=W, N=N)
    return out.reshape(B, N, 3 * C)


# speedup vs baseline: 1.2262x; 1.0358x over previous
import functools

import jax
import jax.numpy as jnp
from jax import lax
from jax.experimental import pallas as pl
from jax.experimental.pallas import tpu as pltpu
from jax.experimental.pallas import tpu_sc as plsc

NC = 2
NS = 16
NW = NC * NS
LANES = 16
Q = 80


def _sc_triplane(table, q0, q1, q2, *, R, C, H, W, N):
    nchunk = R // Q
    iters = (nchunk + NW - 1) // NW
    B = R // N
    CW = C // 2
    mesh = plsc.VectorSubcoreMesh(
        core_axis_name="c", subcore_axis_name="s",
        num_cores=NC, num_subcores=NS)

    picks = ((0, 1), (1, 2), (2, 0))

    @functools.partial(
        pl.kernel,
        out_type=jax.ShapeDtypeStruct((R, 3 * C), jnp.float32),
        mesh=mesh,
        scratch_types=dict(
            q_v=pltpu.VMEM((2, 3, Q), jnp.float32),
            idx_v=pltpu.VMEM((2, 3, 2, Q), jnp.int32),
            w_v=pltpu.VMEM((2, 3, 4, Q), jnp.float32),
            rows=pltpu.VMEM((2, 2, Q, C), jnp.int32),
            out_v=pltpu.VMEM((2, Q, 3 * C), jnp.float32),
            sems=pltpu.SemaphoreType.DMA((2,)),
            osem=pltpu.SemaphoreType.DMA((2,)),
            qsem=pltpu.SemaphoreType.DMA,
        ),
    )
    def k(table_h, q0_h, q1_h, q2_h, out_h, q_v, idx_v, w_v, rows, out_v,
          sems, osem, qsem):
        wid = lax.axis_index("s") * NC + lax.axis_index("c")

        def q_index_stage(ip, it):
            ci = wid + it * NW

            @pl.when(ci < nchunk)
            def _():
                base = ci * Q
                qcps = [pltpu.async_copy(qh.at[pl.ds(base, Q)],
                                         q_v.at[ip, c3], qsem)
                        for c3, qh in enumerate((q0_h, q1_h, q2_h))]
                for qcp in qcps:
                    qcp.wait()
                for p in range(3):
                    px, py = picks[p]
                    for i in range(Q // LANES):
                        sl = pl.ds(i * LANES, LANES)
                        xq = q_v[ip, px, sl]
                        yq = q_v[ip, py, sl]
                        ix = jnp.minimum(jnp.maximum(
                            (xq + 1.0) * (0.5 * (W - 1)), 0.0), float(W - 1))
                        iy = jnp.minimum(jnp.maximum(
                            (yq + 1.0) * (0.5 * (H - 1)), 0.0), float(H - 1))
                        x0 = jnp.minimum(ix.astype(jnp.int32), W - 2)
                        y0 = jnp.minimum(iy.astype(jnp.int32), H - 2)
                        wx = ix - x0.astype(jnp.float32)
                        wy = iy - y0.astype(jnp.float32)
                        ux = 1.0 - wx
                        uy = 1.0 - wy
                        w_v[ip, p, 0, sl] = ux * uy
                        w_v[ip, p, 1, sl] = wx * uy
                        w_v[ip, p, 2, sl] = ux * wy
                        w_v[ip, p, 3, sl] = wx * wy
                        rowid = jnp.arange(LANES, dtype=jnp.int32) + (
                            base + i * LANES)
                        b = jnp.zeros((LANES,), jnp.int32)
                        one = jnp.ones((LANES,), jnp.int32)
                        for bb_ in range(1, B):
                            b = b + jnp.where(rowid >= bb_ * N, one, 0)
                        bi = ((b * 3 + p) * H + y0) * W + x0
                        idx_v[ip, p, 0, sl] = bi
                        idx_v[ip, p, 1, sl] = bi + W

        def fire(ip, p, rbuf):
            for y_ in range(2):
                pltpu.async_copy(table_h.at[idx_v.at[ip, p, y_]],
                                 rows.at[rbuf, y_], sems.at[rbuf])

        def wait_rows(rbuf):
            for y_ in range(2):
                pltpu.make_async_copy(table_h.at[idx_v.at[0, 0, y_]],
                                      rows.at[rbuf, y_],
                                      sems.at[rbuf]).wait()

        def blend_plane(ip, p, rbuf, ob):
            @plsc.parallel_loop(0, Q, step=1, unroll=8)
            def _blend(j):
                grp = j & -LANES
                lidx = jnp.full((LANES,), j & (LANES - 1), dtype=jnp.int32)
                himask = jnp.full((LANES,), -65536, dtype=jnp.int32)
                cw = []
                for t in range(4):
                    wg = w_v[ip, p, t, pl.ds(grp, LANES)]
                    cw.append(wg.at[lidx].get(mode="promise_in_bounds"))
                for g in range(CW // LANES):
                    f_lo = []
                    f_hi = []
                    for y_, off in ((0, 0), (0, CW), (1, 0), (1, CW)):
                        wv = rows[rbuf, y_, j, pl.ds(off + g * LANES, LANES)]
                        f_lo.append(lax.bitcast_convert_type(
                            lax.shift_left(wv, 16), jnp.float32))
                        f_hi.append(lax.bitcast_convert_type(
                            lax.bitwise_and(wv, himask), jnp.float32))
                    for half, f in ((0, f_lo), (1, f_hi)):
                        acc = (f[0] * cw[0] + f[1] * cw[1]
                               + f[2] * cw[2] + f[3] * cw[3])
                        out_v[ob, j,
                              pl.ds(p * C + (g * 2 + half) * LANES, LANES)
                              ] = acc

        def one_chunk(it, ob, bufmap):
            ci = wid + it * NW
            ipc = ob
            ipn = 1 - ob

            @pl.when(ci < nchunk)
            def _():
                base = ci * Q

                @pl.when(it >= 2)
                def _():
                    pltpu.make_async_copy(
                        out_v.at[ob], out_h.at[pl.ds(0, Q)],
                        osem.at[ob]).wait()

                wait_rows(bufmap[0])
                blend_plane(ipc, 0, bufmap[0], ob)
                fire(ipc, 2, bufmap[0])

                q_index_stage(ipn, it + 1)

                wait_rows(bufmap[1])
                blend_plane(ipc, 1, bufmap[1], ob)

                @pl.when(ci + NW < nchunk)
                def _():
                    fire(ipn, 0, bufmap[1])

                wait_rows(bufmap[2])
                blend_plane(ipc, 2, bufmap[2], ob)

                @pl.when(ci + NW < nchunk)
                def _():
                    fire(ipn, 1, bufmap[2])

                pltpu.async_copy(out_v.at[ob], out_h.at[pl.ds(base, Q)],
                                 osem.at[ob])

        def pair_body(it2, _):
            one_chunk(it2 * 2, 0, (0, 1, 0))
            one_chunk(it2 * 2 + 1, 1, (1, 0, 1))
            return 0

        q_index_stage(0, 0)

        @pl.when(wid < nchunk)
        def _():
            fire(0, 0, 0)
            fire(0, 1, 1)

        lax.fori_loop(0, (iters + 1) // 2, pair_body, 0)

        nch = lax.shift_right_logical(nchunk - wid + (NW - 1), 5)

        @pl.when(nch >= 1)
        def _():
            pltpu.make_async_copy(
                out_v.at[0], out_h.at[pl.ds(0, Q)], osem.at[0]).wait()

        @pl.when(nch >= 2)
        def _():
            pltpu.make_async_copy(
                out_v.at[1], out_h.at[pl.ds(0, Q)], osem.at[1]).wait()

    return k(table, q0, q1, q2)


def kernel(rolled_out_feature, query):
    B, C, H, W3 = rolled_out_feature.shape
    W = W3 // 3
    _, N, _ = query.shape
    R = B * N
    table = rolled_out_feature.reshape(B, C, H, 3, W).transpose(0, 3, 2, 4, 1)
    table = table.reshape(B * 3 * H * W, C).astype(jnp.bfloat16)
    table = table.reshape(-1, C // 32, 2, 16).transpose(0, 1, 3, 2)
    t32 = lax.bitcast_convert_type(
        table.reshape(-1, C // 2, 2), jnp.int32)
    t32next = jnp.concatenate([t32[1:], t32[:1]], axis=0)
    table_ov = jnp.concatenate([t32, t32next], axis=1)
    qf = query.reshape(R, 3)
    out = _sc_triplane(table_ov, qf[:, 0], qf[:, 1], qf[:, 2],
                       R=R, C=C, H=H, W=W, N=N)
    return out.reshape(B, N, 3 * C)


# speedup vs baseline: 1.2528x; 1.0217x over previous
import functools

import jax
import jax.numpy as jnp
from jax import lax
from jax.experimental import pallas as pl
from jax.experimental.pallas import tpu as pltpu
from jax.experimental.pallas import tpu_sc as plsc

NC = 2
NS = 16
NW = NC * NS
LANES = 16
Q = 80


def _sc_triplane(table, q0, q1, q2, *, R, C, H, W, N):
    nchunk = R // Q
    iters = (nchunk + NW - 1) // NW
    B = R // N
    CW = C // 2
    mesh = plsc.VectorSubcoreMesh(
        core_axis_name="c", subcore_axis_name="s",
        num_cores=NC, num_subcores=NS)

    picks = ((0, 1), (1, 2), (2, 0))

    @functools.partial(
        pl.kernel,
        out_type=jax.ShapeDtypeStruct((R, 3 * C), jnp.float32),
        mesh=mesh,
        scratch_types=dict(
            q_v=pltpu.VMEM((2, 3, Q), jnp.float32),
            idx_v=pltpu.VMEM((2, 3, 2, Q), jnp.int32),
            w_v=pltpu.VMEM((2, 3, 4, Q), jnp.float32),
            rows=pltpu.VMEM((2, 2, Q, C), jnp.int32),
            out_v=pltpu.VMEM((2, Q, 3 * C), jnp.float32),
            sems=pltpu.SemaphoreType.DMA((2,)),
            osem=pltpu.SemaphoreType.DMA((2,)),
            qsem=pltpu.SemaphoreType.DMA,
        ),
    )
    def k(table_h, q0_h, q1_h, q2_h, out_h, q_v, idx_v, w_v, rows, out_v,
          sems, osem, qsem):
        wid = lax.axis_index("s") * NC + lax.axis_index("c")

        def q_index_stage(ip, it):
            ci = wid + it * NW

            @pl.when(ci < nchunk)
            def _():
                base = ci * Q
                qcps = [pltpu.async_copy(qh.at[pl.ds(base, Q)],
                                         q_v.at[ip, c3], qsem)
                        for c3, qh in enumerate((q0_h, q1_h, q2_h))]
                for qcp in qcps:
                    qcp.wait()
                for p in range(3):
                    px, py = picks[p]
                    for i in range(Q // LANES):
                        sl = pl.ds(i * LANES, LANES)
                        xq = q_v[ip, px, sl]
                        yq = q_v[ip, py, sl]
                        ix = jnp.minimum(jnp.maximum(
                            (xq + 1.0) * (0.5 * (W - 1)), 0.0), float(W - 1))
                        iy = jnp.minimum(jnp.maximum(
                            (yq + 1.0) * (0.5 * (H - 1)), 0.0), float(H - 1))
                        x0 = jnp.minimum(ix.astype(jnp.int32), W - 2)
                        y0 = jnp.minimum(iy.astype(jnp.int32), H - 2)
                        wx = ix - x0.astype(jnp.float32)
                        wy = iy - y0.astype(jnp.float32)
                        ux = 1.0 - wx
                        uy = 1.0 - wy
                        w_v[ip, p, 0, sl] = ux * uy
                        w_v[ip, p, 1, sl] = wx * uy
                        w_v[ip, p, 2, sl] = ux * wy
                        w_v[ip, p, 3, sl] = wx * wy
                        rowid = jnp.arange(LANES, dtype=jnp.int32) + (
                            base + i * LANES)
                        b = jnp.zeros((LANES,), jnp.int32)
                        one = jnp.ones((LANES,), jnp.int32)
                        for bb_ in range(1, B):
                            b = b + jnp.where(rowid >= bb_ * N, one, 0)
                        bi = ((b * 3 + p) * H + y0) * W + x0
                        idx_v[ip, p, 0, sl] = bi
                        idx_v[ip, p, 1, sl] = bi + W

        def fire(ip, p, rbuf):
            for y_ in range(2):
                pltpu.async_copy(table_h.at[idx_v.at[ip, p, y_]],
                                 rows.at[rbuf, y_], sems.at[rbuf])

        def wait_rows(rbuf):
            for y_ in range(2):
                pltpu.make_async_copy(table_h.at[idx_v.at[0, 0, y_]],
                                      rows.at[rbuf, y_],
                                      sems.at[rbuf]).wait()

        def blend_plane(ip, p, rbuf, ob):
            @plsc.parallel_loop(0, Q, step=1, unroll=8)
            def _blend(j):
                grp = j & -LANES
                lidx = jnp.full((LANES,), j & (LANES - 1), dtype=jnp.int32)
                himask = jnp.full((LANES,), -65536, dtype=jnp.int32)
                cw = []
                for t in range(4):
                    wg = w_v[ip, p, t, pl.ds(grp, LANES)]
                    cw.append(wg.at[lidx].get(mode="promise_in_bounds"))
                for g in range(CW // LANES):
                    f_lo = []
                    f_hi = []
                    for y_, off in ((0, 0), (0, CW), (1, 0), (1, CW)):
                        wv = rows[rbuf, y_, j, pl.ds(off + g * LANES, LANES)]
                        f_lo.append(lax.bitcast_convert_type(
                            lax.shift_left(wv, 16), jnp.float32))
                        f_hi.append(lax.bitcast_convert_type(
                            lax.bitwise_and(wv, himask), jnp.float32))
                    for half, f in ((0, f_lo), (1, f_hi)):
                        acc = (f[0] * cw[0] + f[1] * cw[1]
                               + f[2] * cw[2] + f[3] * cw[3])
                        out_v[ob, j,
                              pl.ds(p * C + (g * 2 + half) * LANES, LANES)
                              ] = acc

        def one_chunk(it, ob, bufmap):
            ci = wid + it * NW
            ipc = ob
            ipn = 1 - ob

            @pl.when(ci < nchunk)
            def _():
                base = ci * Q

                @pl.when(it >= 2)
                def _():
                    pltpu.make_async_copy(
                        out_v.at[ob], out_h.at[pl.ds(0, Q)],
                        osem.at[ob]).wait()

                wait_rows(bufmap[0])
                blend_plane(ipc, 0, bufmap[0], ob)
                fire(ipc, 2, bufmap[0])

                q_index_stage(ipn, it + 1)

                wait_rows(bufmap[1])
                blend_plane(ipc, 1, bufmap[1], ob)

                @pl.when(ci + NW < nchunk)
                def _():
                    fire(ipn, 0, bufmap[1])

                wait_rows(bufmap[2])
                blend_plane(ipc, 2, bufmap[2], ob)

                @pl.when(ci + NW < nchunk)
                def _():
                    fire(ipn, 1, bufmap[2])

                pltpu.async_copy(out_v.at[ob], out_h.at[pl.ds(base, Q)],
                                 osem.at[ob])

        def pair_body(it2, _):
            one_chunk(it2 * 2, 0, (0, 1, 0))
            one_chunk(it2 * 2 + 1, 1, (1, 0, 1))
            return 0

        q_index_stage(0, 0)

        @pl.when(wid < nchunk)
        def _():
            fire(0, 0, 0)
            fire(0, 1, 1)

        lax.fori_loop(0, (iters + 1) // 2, pair_body, 0)

        nch = lax.shift_right_logical(nchunk - wid + (NW - 1), 5)

        @pl.when(nch >= 1)
        def _():
            pltpu.make_async_copy(
                out_v.at[0], out_h.at[pl.ds(0, Q)], osem.at[0]).wait()

        @pl.when(nch >= 2)
        def _():
            pltpu.make_async_copy(
                out_v.at[1], out_h.at[pl.ds(0, Q)], osem.at[1]).wait()

    return k(table, q0, q1, q2)


def kernel(rolled_out_feature, query):
    B, C, H, W3 = rolled_out_feature.shape
    W = W3 // 3
    _, N, _ = query.shape
    R = B * N
    CW = C // 2
    W3 = 3 * W
    vi = lax.bitcast_convert_type(rolled_out_feature, jnp.int32)
    r16 = lax.shift_right_logical(
        vi + 0x7FFF + lax.bitwise_and(
            lax.shift_right_logical(vi, 16), 1), 16)
    r16 = r16.reshape(B, C // 32, 2, 16, H, W3)
    words = lax.bitwise_or(r16[:, :, 0],
                           lax.shift_left(r16[:, :, 1], 16))
    words = words.reshape(B, CW, H, 3, W).transpose(0, 3, 2, 4, 1)
    t32 = words.reshape(B * 3 * H * W, CW)
    t32next = jnp.concatenate([t32[1:], t32[:1]], axis=0)
    table_ov = jnp.concatenate([t32, t32next], axis=1)
    qf = query.reshape(R, 3)
    out = _sc_triplane(table_ov, qf[:, 0], qf[:, 1], qf[:, 2],
                       R=R, C=C, H=H, W=W, N=N)
    return out.reshape(B, N, 3 * C)
